# baseline (speedup 1.0000x reference)
import functools
import math

import jax
import jax.numpy as jnp
from jax import lax
from jax.experimental import pallas as pl
from jax.experimental.pallas import tpu as pltpu
from jax.experimental.pallas import tpu_sc as plsc

VOCAB = 100000
D_MODEL = 128
SCALE = math.sqrt(float(D_MODEL))

NUM_CORES = 2
NUM_SUBCORES = 16
NUM_WORKERS = NUM_CORES * NUM_SUBCORES
LANES = 16

B_TOTAL = 1024 * 200
BPW = B_TOTAL // NUM_WORKERS
CHUNK = 128
NCHUNK = BPW // CHUNK
K = 5
NGROUP = NCHUNK // K

_mesh = plsc.VectorSubcoreMesh(core_axis_name="c", subcore_axis_name="s")


@functools.partial(
    pl.kernel,
    out_type=jax.ShapeDtypeStruct((B_TOTAL, D_MODEL), jnp.float32),
    mesh=_mesh,
    scratch_types=[
        pltpu.VMEM((NCHUNK, CHUNK), jnp.int32),
        pltpu.VMEM((K, CHUNK, D_MODEL), jnp.float32),
        pltpu.SemaphoreType.DMA((K,)),
        pltpu.SemaphoreType.DMA((K,)),
    ],
)
def _embed_sc(x_hbm, tbl_hbm, out_hbm, idx_v, rows_v, gsem, wsem):
    wid = lax.axis_index("s") * NUM_CORES + lax.axis_index("c")
    base = wid * BPW

    pltpu.sync_copy(x_hbm.at[wid], idx_v)

    for b in range(K):
        pltpu.make_async_copy(
            tbl_hbm.at[idx_v.at[b]], rows_v.at[b], gsem.at[b]
        ).start()

    def chunk_body(i, carry):
        b = lax.rem(i, K)

        j = i + K - 2

        @pl.when(jnp.logical_and(i >= 2, j < NCHUNK))
        def _refill():
            bj = lax.rem(j, K)
            pltpu.make_async_copy(
                rows_v.at[bj],
                out_hbm.at[pl.ds(base + (j - K) * CHUNK, CHUNK)],
                wsem.at[bj],
            ).wait()
            pltpu.make_async_copy(
                tbl_hbm.at[idx_v.at[j]], rows_v.at[bj], gsem.at[bj]
            ).start()

        pltpu.make_async_copy(
            tbl_hbm.at[idx_v.at[i]], rows_v.at[b], gsem.at[b]
        ).wait()

        def _scale(r, c):
            for rr in range(2):
                for j in range(D_MODEL // LANES):
                    sl = pl.ds(j * LANES, LANES)
                    rows_v[b, 2 * r + rr, sl] = rows_v[b, 2 * r + rr, sl] * SCALE
            return c

        lax.fori_loop(0, CHUNK // 2, _scale, 0)

        pltpu.make_async_copy(
            rows_v.at[b], out_hbm.at[pl.ds(base + i * CHUNK, CHUNK)], wsem.at[b]
        ).start()

        return carry

    lax.fori_loop(0, NCHUNK, chunk_body, 0)

    for b in range(K):
        i = NCHUNK - K + b
        pltpu.make_async_copy(
            rows_v.at[b], out_hbm.at[pl.ds(base + i * CHUNK, CHUNK)], wsem.at[b]
        ).wait()


def kernel(x, emb_weight):
    x_flat = x.reshape(NUM_WORKERS, NCHUNK, CHUNK).astype(jnp.int32)
    out = _embed_sc(x_flat, emb_weight)
    return out.reshape(x.shape + (D_MODEL,))


# speedup vs baseline: 2.8653x; 2.8653x over previous
import functools
import math

import jax
import jax.numpy as jnp
from jax import lax
from jax.experimental import pallas as pl
from jax.experimental.pallas import tpu as pltpu
from jax.experimental.pallas import tpu_sc as plsc

VOCAB = 100000
D_MODEL = 128
SCALE = math.sqrt(float(D_MODEL))

NUM_CORES = 2
NUM_SUBCORES = 16
NUM_WORKERS = NUM_CORES * NUM_SUBCORES
LANES = 16

B_TOTAL = 1024 * 200
BPW = B_TOTAL // NUM_WORKERS
CHUNK = 128
NCHUNK = BPW // CHUNK
K = 5
NGROUP = NCHUNK // K

_mesh = plsc.VectorSubcoreMesh(core_axis_name="c", subcore_axis_name="s")


@functools.partial(
    pl.kernel,
    out_type=jax.ShapeDtypeStruct((B_TOTAL, D_MODEL), jnp.float32),
    mesh=_mesh,
    scratch_types=[
        pltpu.VMEM((NCHUNK, CHUNK), jnp.int32),
        pltpu.VMEM((K, CHUNK, D_MODEL), jnp.float32),
        pltpu.SemaphoreType.DMA((K,)),
        pltpu.SemaphoreType.DMA((K,)),
    ],
)
def _embed_sc(x_hbm, tbl_hbm, out_hbm, idx_v, rows_v, gsem, wsem):
    wid = lax.axis_index("s") * NUM_CORES + lax.axis_index("c")
    base = wid * BPW

    pltpu.sync_copy(x_hbm.at[wid], idx_v)

    for g in range(NGROUP):
        for b in range(K):
            i = g * K + b
            if g > 0:
                pltpu.make_async_copy(
                    rows_v.at[b],
                    out_hbm.at[pl.ds(base + (i - K) * CHUNK, CHUNK)],
                    wsem.at[b],
                ).wait()
            pltpu.make_async_copy(
                tbl_hbm.at[idx_v.at[i]], rows_v.at[b], gsem.at[b]
            ).start()

        for b in range(K):
            i = g * K + b
            pltpu.make_async_copy(
                tbl_hbm.at[idx_v.at[i]], rows_v.at[b], gsem.at[b]
            ).wait()

            def _scale(r, c, b=b):
                for rr in range(2):
                    for j in range(D_MODEL // LANES):
                        sl = pl.ds(j * LANES, LANES)
                        rows_v[b, 2 * r + rr, sl] = rows_v[b, 2 * r + rr, sl] * SCALE
                return c

            lax.fori_loop(0, CHUNK // 2, _scale, 0)

            pltpu.make_async_copy(
                rows_v.at[b], out_hbm.at[pl.ds(base + i * CHUNK, CHUNK)], wsem.at[b]
            ).start()

    for b in range(K):
        i = (NGROUP - 1) * K + b
        pltpu.make_async_copy(
            rows_v.at[b], out_hbm.at[pl.ds(base + i * CHUNK, CHUNK)], wsem.at[b]
        ).wait()


def kernel(x, emb_weight):
    x_flat = x.reshape(NUM_WORKERS, NCHUNK, CHUNK).astype(jnp.int32)
    out = _embed_sc(x_flat, emb_weight)
    return out.reshape(x.shape + (D_MODEL,))


# speedup vs baseline: 2.9927x; 1.0445x over previous
import functools
import math

import jax
import jax.numpy as jnp
from jax import lax
from jax.experimental import pallas as pl
from jax.experimental.pallas import tpu as pltpu
from jax.experimental.pallas import tpu_sc as plsc

VOCAB = 100000
D_MODEL = 128
SCALE = math.sqrt(float(D_MODEL))

NUM_CORES = 2
NUM_SUBCORES = 16
NUM_WORKERS = NUM_CORES * NUM_SUBCORES
LANES = 16

B_TOTAL = 1024 * 200
BPW = B_TOTAL // NUM_WORKERS
CHUNK = 80
NCHUNK = BPW // CHUNK
K = 8
NGROUP = NCHUNK // K

_mesh = plsc.VectorSubcoreMesh(core_axis_name="c", subcore_axis_name="s")


@functools.partial(
    pl.kernel,
    out_type=jax.ShapeDtypeStruct((B_TOTAL, D_MODEL), jnp.float32),
    mesh=_mesh,
    scratch_types=[
        pltpu.VMEM((NCHUNK, CHUNK), jnp.int32),
        pltpu.VMEM((K, CHUNK, D_MODEL), jnp.float32),
        pltpu.SemaphoreType.DMA((K,)),
        pltpu.SemaphoreType.DMA((K,)),
    ],
)
def _embed_sc(x_hbm, tbl_hbm, out_hbm, idx_v, rows_v, gsem, wsem):
    wid = lax.axis_index("s") * NUM_CORES + lax.axis_index("c")
    base = wid * BPW

    pltpu.sync_copy(x_hbm.at[wid], idx_v)

    def group_body(g, carry):
        for b in range(K):
            i = g * K + b

            @pl.when(g > 0)
            def _drain_prev():
                pltpu.make_async_copy(
                    rows_v.at[b],
                    out_hbm.at[pl.ds(base + (i - K) * CHUNK, CHUNK)],
                    wsem.at[b],
                ).wait()

            pltpu.make_async_copy(
                tbl_hbm.at[idx_v.at[i]], rows_v.at[b], gsem.at[b]
            ).start()

        for b in range(K):
            i = g * K + b
            pltpu.make_async_copy(
                tbl_hbm.at[idx_v.at[i]], rows_v.at[b], gsem.at[b]
            ).wait()

            def _scale(r, c, b=b):
                for rr in range(2):
                    for j in range(D_MODEL // LANES):
                        sl = pl.ds(j * LANES, LANES)
                        rows_v[b, 2 * r + rr, sl] = rows_v[b, 2 * r + rr, sl] * SCALE
                return c

            lax.fori_loop(0, CHUNK // 2, _scale, 0)

            pltpu.make_async_copy(
                rows_v.at[b], out_hbm.at[pl.ds(base + i * CHUNK, CHUNK)], wsem.at[b]
            ).start()
        return carry

    lax.fori_loop(0, NGROUP, group_body, 0)

    for b in range(K):
        i = (NGROUP - 1) * K + b
        pltpu.make_async_copy(
            rows_v.at[b], out_hbm.at[pl.ds(base + i * CHUNK, CHUNK)], wsem.at[b]
        ).wait()


def kernel(x, emb_weight):
    x_flat = x.reshape(NUM_WORKERS, NCHUNK, CHUNK).astype(jnp.int32)
    out = _embed_sc(x_flat, emb_weight)
    return out.reshape(x.shape + (D_MODEL,))


# speedup vs baseline: 3.0137x; 1.0070x over previous
import functools
import math

import jax
import jax.numpy as jnp
from jax import lax
from jax.experimental import pallas as pl
from jax.experimental.pallas import tpu as pltpu
from jax.experimental.pallas import tpu_sc as plsc

VOCAB = 100000
D_MODEL = 128
SCALE = math.sqrt(float(D_MODEL))

NUM_CORES = 2
NUM_SUBCORES = 16
NUM_WORKERS = NUM_CORES * NUM_SUBCORES
LANES = 16

B_TOTAL = 1024 * 200
BPW = B_TOTAL // NUM_WORKERS
CHUNK = 64
NCHUNK = BPW // CHUNK
K = 10
NGROUP = NCHUNK // K

_mesh = plsc.VectorSubcoreMesh(core_axis_name="c", subcore_axis_name="s")


@functools.partial(
    pl.kernel,
    out_type=jax.ShapeDtypeStruct((B_TOTAL, D_MODEL), jnp.float32),
    mesh=_mesh,
    scratch_types=[
        pltpu.VMEM((NCHUNK, CHUNK), jnp.int32),
        pltpu.VMEM((K, CHUNK, D_MODEL), jnp.float32),
        pltpu.SemaphoreType.DMA((K,)),
        pltpu.SemaphoreType.DMA((K,)),
    ],
)
def _embed_sc(x_hbm, tbl_hbm, out_hbm, idx_v, rows_v, gsem, wsem):
    wid = lax.axis_index("s") * NUM_CORES + lax.axis_index("c")
    base = wid * BPW

    pltpu.sync_copy(x_hbm.at[wid], idx_v)

    def group_body(g, carry):
        for b in range(K):
            i = g * K + b

            @pl.when(g > 0)
            def _drain_prev():
                pltpu.make_async_copy(
                    rows_v.at[b],
                    out_hbm.at[pl.ds(base + (i - K) * CHUNK, CHUNK)],
                    wsem.at[b],
                ).wait()

            pltpu.make_async_copy(
                tbl_hbm.at[idx_v.at[i]], rows_v.at[b], gsem.at[b]
            ).start()

        for b in range(K):
            i = g * K + b
            pltpu.make_async_copy(
                tbl_hbm.at[idx_v.at[i]], rows_v.at[b], gsem.at[b]
            ).wait()

            def _scale(r, c, b=b):
                for rr in range(2):
                    for j in range(D_MODEL // LANES):
                        sl = pl.ds(j * LANES, LANES)
                        rows_v[b, 2 * r + rr, sl] = rows_v[b, 2 * r + rr, sl] * SCALE
                return c

            lax.fori_loop(0, CHUNK // 2, _scale, 0)

            pltpu.make_async_copy(
                rows_v.at[b], out_hbm.at[pl.ds(base + i * CHUNK, CHUNK)], wsem.at[b]
            ).start()
        return carry

    lax.fori_loop(0, NGROUP, group_body, 0)

    for b in range(K):
        i = (NGROUP - 1) * K + b
        pltpu.make_async_copy(
            rows_v.at[b], out_hbm.at[pl.ds(base + i * CHUNK, CHUNK)], wsem.at[b]
        ).wait()


def kernel(x, emb_weight):
    x_flat = x.reshape(NUM_WORKERS, NCHUNK, CHUNK).astype(jnp.int32)
    out = _embed_sc(x_flat, emb_weight)
    return out.reshape(x.shape + (D_MODEL,))
